# baseline (speedup 1.0000x reference)
import functools

import jax
import jax.numpy as jnp
from jax import lax
from jax.experimental import pallas as pl
from jax.experimental.pallas import tpu as pltpu
from jax.experimental.pallas import tpu_sc as plsc

_BS = 64
_BL = 256
_H = 16

_NPARAM = 96

_IMAGES = ((1, 0, 0), (0, 1, 0), (0, 0, 1),
           (1, 1, 0), (1, 0, 1), (0, 1, 1), (1, 1, 1))

_NW = 32
_SBLK = 2048
_F32 = jnp.float32


def _bf(x):
    return x.astype(jnp.bfloat16).astype(_F32)


def _rne_bits(x):
    b = lax.bitcast_convert_type(x, jnp.uint32)
    b = b + jnp.uint32(0x7FFF) + ((b >> 16) & jnp.uint32(1))
    b = b & jnp.uint32(0xFFFF0000)
    return lax.bitcast_convert_type(b, _F32)


def _tc_body(pr, p_ref, v_ref, m_ref, epa_ref, j_ref):
    g = lambda i: pr[i]
    ic = [g(i) for i in range(3)]
    ce = [g(3 + i) for i in range(3)]
    W1s = [[g(13 + 16 * r + u) for u in range(_H)] for r in range(3)]
    b1s = [g(61 + u) for u in range(_H)]
    W2s = [g(77 + u) for u in range(_H)]
    b2s = g(93)

    v = [v_ref[k] for k in range(3)]
    m = m_ref[...]
    one = jnp.float32(1.0)
    zero = jnp.float32(0.0)

    frac = [_bf(p_ref[k]) * ic[k] for k in range(3)]
    frac = [f - jnp.floor(f) for f in frac]
    w = [_bf(f) * c for f, c in zip(frac, ce)]
    wb = [_bf(x) for x in w]

    cu = [W2s[u] * (W1s[0][u] * v[0] + W1s[1][u] * v[1] + W1s[2][u] * v[2])
          for u in range(_H)]
    ekin = 0.5 * m * (v[0] * v[0] + v[1] * v[1] + v[2] * v[2])

    epot = None
    dedv = None
    for u in range(_H):
        z = wb[0] * W1s[0][u] + wb[1] * W1s[1][u] + wb[2] * W1s[2][u] + b1s[u]
        h = jnp.tanh(z)
        t1 = h * W2s[u]
        t2 = (one - h * h) * cu[u]
        epot = t1 if epot is None else epot + t1
        dedv = t2 if dedv is None else dedv + t2

    etot = (epot + b2s) + ekin
    epa_ref[...] = etot
    conv = [etot * v[j] for j in range(3)]
    vir = [w[j] * dedv for j in range(3)]

    row = jax.lax.broadcasted_iota(jnp.int32, (8, 128), 0)
    col = jax.lax.broadcasted_iota(jnp.int32, (8, 128), 1)
    plane = jnp.zeros((8, 128), jnp.float32)
    vals = [jnp.sum(conv[0]), jnp.sum(conv[1]), jnp.sum(conv[2]),
            jnp.sum(vir[0]), jnp.sum(vir[1]), jnp.sum(vir[2])]
    for k, val in enumerate(vals):
        plane = plane + jnp.where(jnp.logical_and(row == k, col == 0), val, zero)
    j_ref[...] = plane[None]


def _make_sc(Np, N):
    chunk = Np // _NW
    nb = chunk // _SBLK
    nv = _SBLK // 16

    @functools.partial(
        pl.kernel,
        mesh=plsc.VectorSubcoreMesh(core_axis_name="c", subcore_axis_name="s"),
        compiler_params=pltpu.CompilerParams(needs_layout_passes=False),
        out_type=[
            jax.ShapeDtypeStruct((Np,), _F32),
            jax.ShapeDtypeStruct((_NW, 8, 16), _F32),
        ],
        scratch_types=[
            pltpu.VMEM((_NPARAM, 16), _F32),
            pltpu.VMEM((_SBLK,), _F32),
            pltpu.VMEM((_SBLK,), _F32),
            pltpu.VMEM((_SBLK,), _F32),
            pltpu.VMEM((_SBLK,), _F32),
            pltpu.VMEM((_SBLK,), _F32),
            pltpu.VMEM((_SBLK,), _F32),
            pltpu.VMEM((_SBLK,), _F32),
            pltpu.VMEM((_SBLK + 16,), jnp.int32),
            pltpu.VMEM((_SBLK + 16,), _F32),
            pltpu.VMEM((8, 16), _F32),
        ],
    )
    def sc(params_hbm, px_hbm, py_hbm, pz_hbm, vx_hbm, vy_hbm, vz_hbm, m_hbm,
           delta_hbm, jout_hbm,
           pvm, pxv, pyv, pzv, vxv, vyv, vzv, mv, idxv, dlv, jbv):
        wid = lax.axis_index("s") * 2 + lax.axis_index("c")
        base = wid * chunk
        pltpu.sync_copy(params_hbm, pvm)

        P = lambda k: pvm[k]
        ic = [P(i) for i in range(3)]
        ce = [P(3 + i) for i in range(3)]
        nm = [P(6 + i) for i in range(3)]
        hith = [P(9 + i) for i in range(3)]
        cut = P(12)
        W1s = [[P(13 + 16 * r + u) for u in range(_H)] for r in range(3)]
        b1s = [P(61 + u) for u in range(_H)]
        W2s = [P(77 + u) for u in range(_H)]
        b2s = P(93)

        iota = lax.iota(jnp.int32, 16)
        fzero = jnp.zeros((16,), _F32)
        fone = jnp.full((16,), 1.0, _F32)

        def wrap_detect(px16, py16, pz16):
            p3 = (px16, py16, pz16)
            fr = [_rne_bits(p3[k]) * ic[k] for k in range(3)]
            w = [_rne_bits(fr[k]) * ce[k] for k in range(3)]
            wb = [_rne_bits(w[k]) for k in range(3)]
            nc = [wb[k] * nm[k] for k in range(3)]
            lo = [nc[k] <= cut for k in range(3)]
            hi = [nc[k] >= hith[k] for k in range(3)]
            act = [jnp.logical_or(lo[k], hi[k]) for k in range(3)]
            s = [jnp.where(lo[k], fone, fzero) - jnp.where(hi[k], fone, fzero)
                 for k in range(3)]
            return w, act, s

        def block_body(b, jacc):
            off = base + b * _SBLK
            pltpu.sync_copy(px_hbm.at[pl.ds(off, _SBLK)], pxv)
            pltpu.sync_copy(py_hbm.at[pl.ds(off, _SBLK)], pyv)
            pltpu.sync_copy(pz_hbm.at[pl.ds(off, _SBLK)], pzv)
            pltpu.sync_copy(vx_hbm.at[pl.ds(off, _SBLK)], vxv)
            pltpu.sync_copy(vy_hbm.at[pl.ds(off, _SBLK)], vyv)
            pltpu.sync_copy(vz_hbm.at[pl.ds(off, _SBLK)], vzv)
            pltpu.sync_copy(m_hbm.at[pl.ds(off, _SBLK)], mv)

            cap = nv
            ione = jnp.full((16,), 1, jnp.int32)
            izero = jnp.zeros((16,), jnp.int32)

            def det_body(i, ptrv):
                sl = pl.ds(i * 16, 16)
                dlv[sl] = fzero
                _, act, _ = wrap_detect(pxv[sl], pyv[sl], pzv[sl])
                anyc = jnp.logical_or(jnp.logical_or(act[0], act[1]), act[2])
                lidx = iota + i * 16
                anyc = jnp.logical_and(anyc, (lidx + off) < N)
                tgt = jnp.where(anyc, iota * cap + ptrv, _SBLK + iota)
                plsc.store_scatter(idxv, [tgt], lidx)
                return ptrv + jnp.where(anyc, ione, izero)

            ptrv = lax.fori_loop(0, nv, det_body, izero, unroll=False)

            def rep_cond(carry):
                j, _ = carry
                return jnp.any(j < ptrv)

            def rep_body(carry):
                j, jacc = carry
                lanemask = j < ptrv
                idx16 = jnp.where(lanemask,
                                  plsc.load_gather(idxv, [iota * cap + j]), 0)
                pxg = plsc.load_gather(pxv, [idx16])
                pyg = plsc.load_gather(pyv, [idx16])
                pzg = plsc.load_gather(pzv, [idx16])
                vg = [plsc.load_gather(vxv, [idx16]),
                      plsc.load_gather(vyv, [idx16]),
                      plsc.load_gather(vzv, [idx16])]
                mg = plsc.load_gather(mv, [idx16])
                w, act, s = wrap_detect(pxg, pyg, pzg)
                A = [s[k] * ce[k] for k in range(3)]
                ekin = 0.5 * mg * (vg[0] * vg[0] + vg[1] * vg[1] + vg[2] * vg[2])
                cu = [W2s[u] * (W1s[0][u] * vg[0] + W1s[1][u] * vg[1]
                                + W1s[2][u] * vg[2]) for u in range(_H)]
                for bits in _IMAGES:
                    valid = lanemask
                    for a in range(3):
                        if bits[a]:
                            valid = jnp.logical_and(valid, act[a])
                    pimg = [w[k] + A[k] if bits[k] else w[k] for k in range(3)]
                    pb = [_rne_bits(x) for x in pimg]
                    epot = None
                    dedv = None
                    for u in range(_H):
                        z = (pb[0] * W1s[0][u] + pb[1] * W1s[1][u]
                             + pb[2] * W1s[2][u] + b1s[u])
                        e2 = jnp.exp(z + z)
                        h = fone - 2.0 / (e2 + fone)
                        t1 = h * W2s[u]
                        t2 = (fone - h * h) * cu[u]
                        epot = t1 if epot is None else epot + t1
                        dedv = t2 if dedv is None else dedv + t2
                    etot = jnp.where(valid, (epot + b2s) + ekin, fzero)
                    dedvm = jnp.where(valid, dedv, fzero)
                    idx16t = jnp.where(valid, idx16, _SBLK + iota)
                    plsc.addupdate_scatter(dlv, [idx16t], etot)
                    jacc = (jacc[0] + etot * vg[0], jacc[1] + etot * vg[1],
                            jacc[2] + etot * vg[2], jacc[3] + pimg[0] * dedvm,
                            jacc[4] + pimg[1] * dedvm, jacc[5] + pimg[2] * dedvm)
                return (j + 1, jacc)

            _, jacc = lax.while_loop(rep_cond, rep_body, (jnp.int32(0), jacc))
            pltpu.sync_copy(dlv.at[pl.ds(0, _SBLK)], delta_hbm.at[pl.ds(off, _SBLK)])
            return jacc

        jacc0 = (fzero,) * 6
        jacc = lax.fori_loop(0, nb, block_body, jacc0, unroll=False)
        for k in range(6):
            jbv[k] = jacc[k]
        jbv[6] = fzero
        jbv[7] = fzero
        pltpu.sync_copy(jbv, jout_hbm.at[wid])

    return sc


def kernel(positions, cell, types, masses, velocities, W1, b1, W2, b2, cutoff):
    del types
    f32 = jnp.float32
    N = positions.shape[0]
    cell = cell.astype(f32)
    inv_cell = jnp.linalg.inv(cell)
    recip = inv_cell.T
    norms = jnp.linalg.norm(recip, axis=1)
    heights = 1.0 / norms
    normals = recip / norms[:, None]
    cut = jnp.asarray(cutoff, f32)

    dg = lambda x: jnp.diagonal(x)
    params = jnp.concatenate([
        _rne_bits(dg(inv_cell).astype(f32)), _rne_bits(dg(cell).astype(f32)),
        _rne_bits(dg(normals).astype(f32)),
        (heights - cut).reshape(-1), cut.reshape(1),
        _rne_bits(W1.astype(f32)).reshape(-1), b1.astype(f32).reshape(-1),
        _rne_bits(W2.astype(f32)).reshape(-1), b2.astype(f32).reshape(-1),
    ])
    params = jnp.concatenate([params, jnp.zeros((_NPARAM - params.shape[0],), f32)])

    unit = _NW * _SBLK
    Np = ((N + unit - 1) // unit) * unit
    G = Np // (_BS * _BL)
    R = Np // _BL
    pad = Np - N
    pos_p = jnp.pad(positions.astype(f32), ((0, pad), (0, 0)))
    vel_p = jnp.pad(velocities.astype(f32), ((0, pad), (0, 0)))
    m_p = jnp.pad(masses[:, 0].astype(f32), (0, pad))
    pos_t = pos_p.T.reshape(3, R, _BL)
    vel_t = vel_p.T.reshape(3, R, _BL)
    m_t = m_p.reshape(R, _BL)

    epa_o, jp = pl.pallas_call(
        _tc_body,
        grid=(G,),
        in_specs=[
            pl.BlockSpec(memory_space=pltpu.SMEM),
            pl.BlockSpec((3, _BS, _BL), lambda i: (0, i, 0)),
            pl.BlockSpec((3, _BS, _BL), lambda i: (0, i, 0)),
            pl.BlockSpec((_BS, _BL), lambda i: (i, 0)),
        ],
        out_specs=[
            pl.BlockSpec((_BS, _BL), lambda i: (i, 0)),
            pl.BlockSpec((1, 8, 128), lambda i: (i, 0, 0)),
        ],
        out_shape=[
            jax.ShapeDtypeStruct((R, _BL), f32),
            jax.ShapeDtypeStruct((G, 8, 128), f32),
        ],
        compiler_params=pltpu.CompilerParams(
            dimension_semantics=("parallel",),
        ),
    )(params, pos_t, vel_t, m_t)

    params_rep = jnp.broadcast_to(params[:, None], (_NPARAM, 16))
    delta, jsc = _make_sc(Np, N)(
        params_rep,
        pos_t[0].reshape(Np), pos_t[1].reshape(Np), pos_t[2].reshape(Np),
        vel_t[0].reshape(Np), vel_t[1].reshape(Np), vel_t[2].reshape(Np),
        m_p,
    )

    e_per_atom = (epa_o.reshape(Np) + delta)[:N]
    js = jp.sum(axis=0)
    js2 = jsc.sum(axis=(0, 2))
    J = (js[0:3, 0] + js2[0:3]) - (js[3:6, 0] + js2[3:6])
    return (J, e_per_atom)


# speedup vs baseline: 2.2376x; 2.2376x over previous
import functools

import jax
import jax.numpy as jnp
from jax import lax
from jax.experimental import pallas as pl
from jax.experimental.pallas import tpu as pltpu
from jax.experimental.pallas import tpu_sc as plsc

_BS = 64
_BL = 256
_H = 16

_NPARAM = 112

_IMAGES = ((1, 0, 0), (0, 1, 0), (0, 0, 1),
           (1, 1, 0), (1, 0, 1), (0, 1, 1), (1, 1, 1))

_NW = 32
_SBLK = 8192
_CAP_S = 160
_CAP_M = 64
_F32 = jnp.float32


def _bf(x):
    return x.astype(jnp.bfloat16).astype(_F32)


def _rne_bits(x):
    b = lax.bitcast_convert_type(x, jnp.uint32)
    b = b + jnp.uint32(0x7FFF) + ((b >> 16) & jnp.uint32(1))
    b = b & jnp.uint32(0xFFFF0000)
    return lax.bitcast_convert_type(b, _F32)


def _tc_body(pr, d_ref, epa_ref, j_ref):
    g = lambda i: pr[i]
    ic = [g(i) for i in range(3)]
    ce = [g(3 + i) for i in range(3)]
    W1s = [[g(13 + 16 * r + u) for u in range(_H)] for r in range(3)]
    b1s = [g(61 + u) for u in range(_H)]
    W2s = [g(77 + u) for u in range(_H)]
    b2s = g(93)

    v = [d_ref[3 + k] for k in range(3)]
    m = d_ref[6]
    one = jnp.float32(1.0)
    zero = jnp.float32(0.0)

    frac = [_bf(d_ref[k]) * ic[k] for k in range(3)]
    frac = [f - jnp.floor(f) for f in frac]
    w = [_bf(f) * c for f, c in zip(frac, ce)]
    wb = [_bf(x) for x in w]

    cu = [W2s[u] * (W1s[0][u] * v[0] + W1s[1][u] * v[1] + W1s[2][u] * v[2])
          for u in range(_H)]
    ekin = 0.5 * m * (v[0] * v[0] + v[1] * v[1] + v[2] * v[2])

    epot = None
    dedv = None
    for u in range(_H):
        z = wb[0] * W1s[0][u] + wb[1] * W1s[1][u] + wb[2] * W1s[2][u] + b1s[u]
        h = jnp.tanh(z)
        t1 = h * W2s[u]
        t2 = (one - h * h) * cu[u]
        epot = t1 if epot is None else epot + t1
        dedv = t2 if dedv is None else dedv + t2

    etot = (epot + b2s) + ekin
    epa_ref[...] = etot
    conv = [etot * v[j] for j in range(3)]
    vir = [w[j] * dedv for j in range(3)]

    row = jax.lax.broadcasted_iota(jnp.int32, (8, 128), 0)
    col = jax.lax.broadcasted_iota(jnp.int32, (8, 128), 1)
    plane = jnp.zeros((8, 128), jnp.float32)
    vals = [jnp.sum(conv[0]), jnp.sum(conv[1]), jnp.sum(conv[2]),
            jnp.sum(vir[0]), jnp.sum(vir[1]), jnp.sum(vir[2])]
    for k, val in enumerate(vals):
        plane = plane + jnp.where(jnp.logical_and(row == k, col == 0), val, zero)
    j_ref[...] = plane[None]


def _make_sc(Np, N):
    chunk = Np // _NW
    nb = chunk // _SBLK
    nv = _SBLK // 16

    @functools.partial(
        pl.kernel,
        mesh=plsc.VectorSubcoreMesh(core_axis_name="c", subcore_axis_name="s"),
        compiler_params=pltpu.CompilerParams(needs_layout_passes=False),
        out_type=[
            jax.ShapeDtypeStruct((Np,), _F32),
            jax.ShapeDtypeStruct((_NW, 8, 16), _F32),
        ],
        scratch_types=[
            pltpu.VMEM((_NPARAM, 16), _F32),
            pltpu.VMEM((8, _SBLK), _F32),
            pltpu.VMEM((16 * _CAP_S + 16,), jnp.int32),
            pltpu.VMEM((16 * _CAP_M + 16,), jnp.int32),
            pltpu.VMEM((_SBLK + 16,), _F32),
            pltpu.VMEM((8, 16), _F32),
        ],
    )
    def sc(params_hbm, data_hbm, delta_hbm, jout_hbm,
           pvm, dv, idxs, idxm, dlv, jbv):
        wid = lax.axis_index("s") * 2 + lax.axis_index("c")
        base = wid * chunk
        pltpu.sync_copy(params_hbm, pvm)

        P = lambda k: pvm[k]
        ce = [P(3 + i) for i in range(3)]
        ic = [P(i) for i in range(3)]
        W1s = [[P(13 + 16 * r + u) for u in range(_H)] for r in range(3)]
        b1s = [P(61 + u) for u in range(_H)]
        W2s = [P(77 + u) for u in range(_H)]
        b2s = P(93)
        plo = [P(94 + i) for i in range(3)]
        phi = [P(97 + i) for i in range(3)]

        iota = lax.iota(jnp.int32, 16)
        fzero = jnp.zeros((16,), _F32)
        fone = jnp.full((16,), 1.0, _F32)
        ione = jnp.full((16,), 1, jnp.int32)
        izero = jnp.zeros((16,), jnp.int32)

        def mlp(pimg, cu, valid, ekin):
            pb = [_rne_bits(x) for x in pimg]
            epot = None
            dedv = None
            for u in range(_H):
                z = (pb[0] * W1s[0][u] + pb[1] * W1s[1][u]
                     + pb[2] * W1s[2][u] + b1s[u])
                e2 = jnp.exp(z + z)
                h = fone - 2.0 / (e2 + fone)
                t1 = h * W2s[u]
                t2 = (fone - h * h) * cu[u]
                epot = t1 if epot is None else epot + t1
                dedv = t2 if dedv is None else dedv + t2
            etot = jnp.where(valid, (epot + b2s) + ekin, fzero)
            dedvm = jnp.where(valid, dedv, fzero)
            return etot, dedvm

        def gather_atom(idx16):
            pg = [plsc.load_gather(dv, [jnp.full((16,), k, jnp.int32), idx16])
                  for k in range(3)]
            vg = [plsc.load_gather(dv, [jnp.full((16,), 3 + k, jnp.int32), idx16])
                  for k in range(3)]
            mg = plsc.load_gather(dv, [jnp.full((16,), 6, jnp.int32), idx16])
            lo = [pg[k] <= plo[k] for k in range(3)]
            hi = [pg[k] >= phi[k] for k in range(3)]
            act = [jnp.logical_or(lo[k], hi[k]) for k in range(3)]
            s = [jnp.where(lo[k], fone, fzero) - jnp.where(hi[k], fone, fzero)
                 for k in range(3)]
            fr = [_rne_bits(pg[k]) * ic[k] for k in range(3)]
            w = [_rne_bits(fr[k]) * ce[k] for k in range(3)]
            A = [s[k] * ce[k] for k in range(3)]
            ekin = 0.5 * mg * (vg[0] * vg[0] + vg[1] * vg[1] + vg[2] * vg[2])
            cu = [W2s[u] * (W1s[0][u] * vg[0] + W1s[1][u] * vg[1]
                            + W1s[2][u] * vg[2]) for u in range(_H)]
            return vg, act, s, w, A, ekin, cu

        def block_body(b, jacc):
            off = base + b * _SBLK
            pltpu.sync_copy(data_hbm.at[:, pl.ds(off, _SBLK)], dv)

            def det_body(i, ptrs):
                ptr_s, ptr_m = ptrs
                sl = pl.ds(i * 16, 16)
                dlv[sl] = fzero
                px = dv[0, sl]
                py = dv[1, sl]
                pz = dv[2, sl]
                acts = [jnp.logical_or(px <= plo[0], px >= phi[0]),
                        jnp.logical_or(py <= plo[1], py >= phi[1]),
                        jnp.logical_or(pz <= plo[2], pz >= phi[2])]
                na = (jnp.where(acts[0], ione, izero)
                      + jnp.where(acts[1], ione, izero)
                      + jnp.where(acts[2], ione, izero))
                lidx = iota + i * 16
                inb = (lidx + off) < N
                one_ax = jnp.logical_and(na == 1, inb)
                multi = jnp.logical_and(na >= 2, inb)
                tgt_s = jnp.where(one_ax, iota * _CAP_S + ptr_s, 16 * _CAP_S + iota)
                plsc.store_scatter(idxs, [tgt_s], lidx)
                tgt_m = jnp.where(multi, iota * _CAP_M + ptr_m, 16 * _CAP_M + iota)
                plsc.store_scatter(idxm, [tgt_m], lidx)
                return (ptr_s + jnp.where(one_ax, ione, izero),
                        ptr_m + jnp.where(multi, ione, izero))

            ptr_s, ptr_m = lax.fori_loop(0, nv, det_body, (izero, izero),
                                         unroll=False)

            def s_cond(carry):
                j, _ = carry
                return jnp.any(j < ptr_s)

            def s_body(carry):
                j, jacc = carry
                lanemask = j < ptr_s
                idx16 = jnp.where(lanemask,
                                  plsc.load_gather(idxs, [iota * _CAP_S + j]), 0)
                vg, _act, _s, w, A, ekin, cu = gather_atom(idx16)
                pimg = [w[k] + A[k] for k in range(3)]
                etot, dedvm = mlp(pimg, cu, lanemask, ekin)
                idx16t = jnp.where(lanemask, idx16, _SBLK + iota)
                plsc.addupdate_scatter(dlv, [idx16t], etot)
                jacc = (jacc[0] + etot * vg[0], jacc[1] + etot * vg[1],
                        jacc[2] + etot * vg[2], jacc[3] + pimg[0] * dedvm,
                        jacc[4] + pimg[1] * dedvm, jacc[5] + pimg[2] * dedvm)
                return (j + 1, jacc)

            j0 = jnp.int32(0)
            _, jacc = lax.while_loop(s_cond, s_body, (j0, jacc))

            def m_cond(carry):
                j, _ = carry
                return jnp.any(j < ptr_m)

            def m_body(carry):
                j, jacc = carry
                lanemask = j < ptr_m
                idx16 = jnp.where(lanemask,
                                  plsc.load_gather(idxm, [iota * _CAP_M + j]), 0)
                vg, act, _s, w, A, ekin, cu = gather_atom(idx16)
                idx16t = jnp.where(lanemask, idx16, _SBLK + iota)
                for bits in _IMAGES:
                    valid = lanemask
                    for a in range(3):
                        if bits[a]:
                            valid = jnp.logical_and(valid, act[a])
                    pimg = [w[k] + A[k] if bits[k] else w[k] for k in range(3)]
                    etot, dedvm = mlp(pimg, cu, valid, ekin)
                    plsc.addupdate_scatter(dlv, [idx16t], etot)
                    jacc = (jacc[0] + etot * vg[0], jacc[1] + etot * vg[1],
                            jacc[2] + etot * vg[2], jacc[3] + pimg[0] * dedvm,
                            jacc[4] + pimg[1] * dedvm, jacc[5] + pimg[2] * dedvm)
                return (j + 1, jacc)

            _, jacc = lax.while_loop(m_cond, m_body, (j0, jacc))

            pltpu.sync_copy(dlv.at[pl.ds(0, _SBLK)], delta_hbm.at[pl.ds(off, _SBLK)])
            return jacc

        jacc0 = (fzero,) * 6
        jacc = lax.fori_loop(0, nb, block_body, jacc0, unroll=False)
        for k in range(6):
            jbv[k] = jacc[k]
        jbv[6] = fzero
        jbv[7] = fzero
        pltpu.sync_copy(jbv, jout_hbm.at[wid])

    return sc


def kernel(positions, cell, types, masses, velocities, W1, b1, W2, b2, cutoff):
    del types
    f32 = jnp.float32
    N = positions.shape[0]
    cell = cell.astype(f32)
    inv_cell = jnp.linalg.inv(cell)
    recip = inv_cell.T
    norms = jnp.linalg.norm(recip, axis=1)
    heights = 1.0 / norms
    normals = recip / norms[:, None]
    cut = jnp.asarray(cutoff, f32)

    dg = lambda x: jnp.diagonal(x)
    icd = _rne_bits(dg(inv_cell).astype(f32))
    ced = _rne_bits(dg(cell).astype(f32))
    nmd = _rne_bits(dg(normals).astype(f32))
    hith = (heights - cut).astype(f32)

    def chain(pbits):
        p = lax.bitcast_convert_type(pbits, f32)
        fr = _rne_bits(p) * icd
        w = _rne_bits(fr) * ced
        return _rne_bits(w) * nmd

    maxb = jnp.full((3,), 0x7F7FFFFF, jnp.int32)
    lo_b, hi_b = jnp.zeros((3,), jnp.int32), maxb
    for _ in range(34):
        mid = lo_b + (hi_b - lo_b) // 2
        c = chain(mid) <= cut
        lo_b = jnp.where(c, mid, lo_b)
        hi_b = jnp.where(c, hi_b, mid)
    p_lo = lax.bitcast_convert_type(lo_b, f32)
    lo_b, hi_b = jnp.zeros((3,), jnp.int32), maxb
    for _ in range(34):
        mid = lo_b + (hi_b - lo_b) // 2
        c = chain(mid) >= hith
        hi_b = jnp.where(c, mid, hi_b)
        lo_b = jnp.where(c, lo_b, mid)
    p_hi = lax.bitcast_convert_type(hi_b, f32)

    params = jnp.concatenate([
        icd, ced, nmd, hith.reshape(-1), cut.reshape(1),
        _rne_bits(W1.astype(f32)).reshape(-1), b1.astype(f32).reshape(-1),
        _rne_bits(W2.astype(f32)).reshape(-1), b2.astype(f32).reshape(-1),
        p_lo, p_hi,
    ])
    params = jnp.concatenate([params, jnp.zeros((_NPARAM - params.shape[0],), f32)])

    unit = _NW * _SBLK
    Np = ((N + unit - 1) // unit) * unit
    G = Np // (_BS * _BL)
    R = Np // _BL
    pad = Np - N
    pos_p = jnp.pad(positions.astype(f32), ((0, pad), (0, 0)))
    vel_p = jnp.pad(velocities.astype(f32), ((0, pad), (0, 0)))
    m_p = jnp.pad(masses[:, 0].astype(f32), (0, pad))
    data8 = jnp.concatenate([pos_p.T, vel_p.T, m_p[None], jnp.zeros((1, Np), f32)])

    epa_o, jp = pl.pallas_call(
        _tc_body,
        grid=(G,),
        in_specs=[
            pl.BlockSpec(memory_space=pltpu.SMEM),
            pl.BlockSpec((8, _BS, _BL), lambda i: (0, i, 0)),
        ],
        out_specs=[
            pl.BlockSpec((_BS, _BL), lambda i: (i, 0)),
            pl.BlockSpec((1, 8, 128), lambda i: (i, 0, 0)),
        ],
        out_shape=[
            jax.ShapeDtypeStruct((R, _BL), f32),
            jax.ShapeDtypeStruct((G, 8, 128), f32),
        ],
        compiler_params=pltpu.CompilerParams(
            dimension_semantics=("parallel",),
        ),
    )(params, data8.reshape(8, R, _BL))

    params_rep = jnp.broadcast_to(params[:, None], (_NPARAM, 16))
    delta, jsc = _make_sc(Np, N)(params_rep, data8)

    e_per_atom = (epa_o.reshape(Np) + delta)[:N]
    js = jp.sum(axis=0)
    js2 = jsc.sum(axis=(0, 2))
    J = (js[0:3, 0] + js2[0:3]) - (js[3:6, 0] + js2[3:6])
    return (J, e_per_atom)
